# baseline (speedup 1.0000x reference)
import functools

import jax
import jax.numpy as jnp
from jax import lax
from jax.experimental import pallas as pl
from jax.experimental.pallas import tpu as pltpu
from jax.experimental.pallas import tpu_sc as plsc

D_MODEL = 64
CHUNK = 128


@functools.lru_cache(maxsize=None)
def _make_gather(n_total: int, vocab: int, d: int):
    info = plsc.get_sparse_core_info()
    nc, ns = info.num_cores, info.num_subcores
    nw = nc * ns
    assert n_total % (nw * CHUNK) == 0
    n_per_w = n_total // nw
    n_chunks = n_per_w // CHUNK

    mesh = plsc.VectorSubcoreMesh(core_axis_name="c", subcore_axis_name="s")

    @functools.partial(
        pl.kernel,
        mesh=mesh,
        compiler_params=pltpu.CompilerParams(use_tc_tiling_on_sc=False),
        out_type=jax.ShapeDtypeStruct((n_total, d), jnp.float32),
        scratch_types=[
            pltpu.VMEM((n_per_w,), jnp.int32),
            pltpu.VMEM((CHUNK, d), jnp.float32),
            pltpu.SemaphoreType.DMA,
        ],
    )
    def gather_kernel(idx_hbm, table_hbm, out_hbm, idx_all, rows, gsem):
        wid = lax.axis_index("s") * nc + lax.axis_index("c")
        base = wid * n_per_w
        pltpu.sync_copy(idx_hbm.at[pl.ds(base, n_per_w)], idx_all)

        def body(g, carry):
            off = g * CHUNK
            pltpu.async_copy(
                table_hbm.at[idx_all.at[pl.ds(off, CHUNK)]], rows, gsem
            ).wait()
            pltpu.sync_copy(rows, out_hbm.at[pl.ds(base + off, CHUNK)])
            return carry

        lax.fori_loop(0, n_chunks, body, 0)

    return gather_kernel


def kernel(x, table):
    b, l = x.shape
    vocab, d = table.shape
    flat = x.reshape(b * l).astype(jnp.int32)
    out = _make_gather(b * l, vocab, d)(flat, table)
    return out.reshape(b, l, d)


# speedup vs baseline: 1.1169x; 1.1169x over previous
import functools

import jax
import jax.numpy as jnp
from jax import lax
from jax.experimental import pallas as pl
from jax.experimental.pallas import tpu as pltpu
from jax.experimental.pallas import tpu_sc as plsc

D_MODEL = 64
CHUNK = 128
NBUF = 4
GROUP = CHUNK * NBUF


@functools.lru_cache(maxsize=None)
def _make_gather(n_total: int, vocab: int, d: int):
    info = plsc.get_sparse_core_info()
    nc, ns = info.num_cores, info.num_subcores
    nw = nc * ns
    assert n_total % (nw * GROUP) == 0
    n_per_w = n_total // nw
    n_groups = n_per_w // GROUP
    assert n_groups % 2 == 0

    mesh = plsc.VectorSubcoreMesh(core_axis_name="c", subcore_axis_name="s")

    @functools.partial(
        pl.kernel,
        mesh=mesh,
        compiler_params=pltpu.CompilerParams(use_tc_tiling_on_sc=False),
        out_type=jax.ShapeDtypeStruct((n_total, d), jnp.float32),
        scratch_types=[
            pltpu.VMEM((n_per_w,), jnp.int32),
            pltpu.VMEM((2, GROUP, d), jnp.float32),
            pltpu.SemaphoreType.DMA,
            pltpu.SemaphoreType.DMA,
            pltpu.SemaphoreType.DMA,
            pltpu.SemaphoreType.DMA,
        ],
    )
    def gather_kernel(idx_hbm, table_hbm, out_hbm, idx_all, rows, g0, g1, s0, s1):
        wid = lax.axis_index("s") * nc + lax.axis_index("c")
        base = wid * n_per_w
        gsem = (g0, g1)
        ssem = (s0, s1)
        pltpu.sync_copy(idx_hbm.at[pl.ds(base, n_per_w)], idx_all)

        def gstart(p, g):
            for b in range(NBUF):
                pltpu.async_copy(
                    table_hbm.at[idx_all.at[pl.ds(g * GROUP + b * CHUNK, CHUNK)]],
                    rows.at[p, pl.ds(b * CHUNK, CHUNK)],
                    gsem[p],
                )

        def gwait(p):
            for b in range(NBUF):
                pltpu.make_async_copy(
                    table_hbm.at[idx_all.at[pl.ds(b * CHUNK, CHUNK)]],
                    rows.at[p, pl.ds(b * CHUNK, CHUNK)],
                    gsem[p],
                ).wait()

        def sstart(p, g):
            pltpu.async_copy(
                rows.at[p], out_hbm.at[pl.ds(base + g * GROUP, GROUP)], ssem[p]
            )

        def swait(p):
            pltpu.make_async_copy(
                rows.at[p], out_hbm.at[pl.ds(base, GROUP)], ssem[p]
            ).wait()

        def handle(g, p):
            pl.when(g > 0)(lambda: swait(1 - p))
            pl.when(g + 1 < n_groups)(lambda: gstart(1 - p, g + 1))
            gwait(p)
            sstart(p, g)

        gstart(0, 0)

        def body(i2, carry):
            handle(2 * i2, 0)
            handle(2 * i2 + 1, 1)
            return carry

        lax.fori_loop(0, n_groups // 2, body, 0)
        swait((n_groups - 1) % 2)

    return gather_kernel


def kernel(x, table):
    b, l = x.shape
    vocab, d = table.shape
    flat = x.reshape(b * l).astype(jnp.int32)
    out = _make_gather(b * l, vocab, d)(flat, table)
    return out.reshape(b, l, d)


# speedup vs baseline: 1.3581x; 1.2160x over previous
import functools

import jax
import jax.numpy as jnp
from jax import lax
from jax.experimental import pallas as pl
from jax.experimental.pallas import tpu as pltpu
from jax.experimental.pallas import tpu_sc as plsc

CHUNK = 128
NBUF = 2
GROUP = CHUNK * NBUF


@functools.lru_cache(maxsize=None)
def _make_gather(n_total: int, vocab: int, dpad: int, d: int):
    info = plsc.get_sparse_core_info()
    nc, ns = info.num_cores, info.num_subcores
    nw = nc * ns
    n_per_w = n_total // nw
    n_groups = n_per_w // GROUP

    mesh = plsc.VectorSubcoreMesh(core_axis_name="c", subcore_axis_name="s")

    @functools.partial(
        pl.kernel,
        mesh=mesh,
        compiler_params=pltpu.CompilerParams(use_tc_tiling_on_sc=True),
        out_type=jax.ShapeDtypeStruct((n_total, dpad), jnp.float32),
        scratch_types=[
            pltpu.VMEM((n_per_w,), jnp.int32),
            pltpu.VMEM((2, GROUP, dpad), jnp.float32),
            pltpu.SemaphoreType.DMA,
            pltpu.SemaphoreType.DMA,
            pltpu.SemaphoreType.DMA,
            pltpu.SemaphoreType.DMA,
        ],
    )
    def gather_kernel(idx_hbm, table_hbm, out_hbm, idx_all, rows, g0, g1, s0, s1):
        wid = lax.axis_index("s") * nc + lax.axis_index("c")
        base = wid * n_per_w
        gsem = (g0, g1)
        ssem = (s0, s1)
        pltpu.sync_copy(idx_hbm.at[pl.ds(base, n_per_w)], idx_all)

        def gstart(p, g):
            for b in range(NBUF):
                pltpu.async_copy(
                    table_hbm.at[idx_all.at[pl.ds(g * GROUP + b * CHUNK, CHUNK)]],
                    rows.at[p, pl.ds(b * CHUNK, CHUNK)],
                    gsem[p],
                )

        def gwait(p):
            for b in range(NBUF):
                pltpu.make_async_copy(
                    table_hbm.at[idx_all.at[pl.ds(b * CHUNK, CHUNK)]],
                    rows.at[p, pl.ds(b * CHUNK, CHUNK)],
                    gsem[p],
                ).wait()

        def sstart(p, g):
            pltpu.async_copy(
                rows.at[p],
                out_hbm.at[pl.ds(base + g * GROUP, GROUP)],
                ssem[p],
            )

        def swait(p):
            pltpu.make_async_copy(
                rows.at[p],
                out_hbm.at[pl.ds(base, GROUP)],
                ssem[p],
            ).wait()

        def handle(g, p):
            pl.when(g > 0)(lambda: swait(1 - p))
            pl.when(g + 1 < n_groups)(lambda: gstart(1 - p, g + 1))
            gwait(p)
            sstart(p, g)

        gstart(0, 0)

        def body(i2, carry):
            handle(2 * i2, 0)
            handle(2 * i2 + 1, 1)
            return carry

        lax.fori_loop(0, n_groups // 2, body, 0)
        swait((n_groups - 1) % 2)

    return gather_kernel


def kernel(x, table):
    b, l = x.shape
    vocab, d = table.shape
    dpad = 128
    table_pad = jnp.pad(table, ((0, 0), (0, dpad - d)))
    flat = x.reshape(b * l).astype(jnp.int32)
    out = _make_gather(b * l, vocab, dpad, d)(flat, table_pad)
    return out[:, :d].reshape(b, l, d)
